# baseline (speedup 1.0000x reference)
import functools

import jax
import jax.numpy as jnp
from jax import lax
from jax.experimental import pallas as pl
from jax.experimental.pallas import tpu as pltpu
from jax.experimental.pallas import tpu_sc as plsc

B = 16384
D = 64
NC = 2
NS = 16
NW = NC * NS
BW = B // NW
L = 16
NG = BW // L
BCH = 128
NBCH = BW // BCH
HALF = BW // 2

_mesh = plsc.VectorSubcoreMesh(core_axis_name="c", subcore_axis_name="s")


@functools.partial(
    pl.kernel,
    mesh=_mesh,
    compiler_params=pltpu.CompilerParams(needs_layout_passes=False),
    out_type=jax.ShapeDtypeStruct((B,), jnp.float32),
    scratch_types=[
        pltpu.VMEM((BW,), jnp.int32),
        pltpu.VMEM((BW,), jnp.int32),
        pltpu.VMEM((HALF, D), jnp.float32),
        pltpu.VMEM((HALF, D), jnp.float32),
        pltpu.VMEM((BCH, 1), jnp.float32),
        pltpu.VMEM((BCH, 1), jnp.float32),
        pltpu.VMEM((BW,), jnp.float32),
        pltpu.VMEM((BW,), jnp.float32),
        pltpu.SemaphoreType.DMA,
        pltpu.SemaphoreType.DMA,
    ],
)
def _mf_sc(u_hbm, i_hbm, ue_hbm, ie_hbm, ub_hbm, ib_hbm, out_hbm,
           u_v, i_v, ue2, ie2, ub2, ib2, bias_v, o_v, sem, bsem):
    wid = lax.axis_index("s") * NC + lax.axis_index("c")
    base = wid * BW

    pltpu.sync_copy(u_hbm.at[pl.ds(base, BW)], u_v)
    pltpu.sync_copy(i_hbm.at[pl.ds(base, BW)], i_v)

    zeros = jnp.zeros((L,), jnp.int32)
    iota = lax.iota(jnp.int32, L)

    def bias_fire(g, _):
        uvec = u_v[pl.ds(g * L, L)]
        ivec = i_v[pl.ds(g * L, L)]
        for l in range(L):
            dst = pl.ds((g % (BCH // L)) * L + l, 1)
            pltpu.async_copy(ub_hbm.at[pl.ds(uvec[l], 1)], ub2.at[dst], bsem)
            pltpu.async_copy(ib_hbm.at[pl.ds(ivec[l], 1)], ib2.at[dst], bsem)
        return 0

    def bias_drain(g, _):
        uvec = u_v[pl.ds(g * L, L)]
        ivec = i_v[pl.ds(g * L, L)]
        for l in range(L):
            dst = pl.ds((g % (BCH // L)) * L + l, 1)
            pltpu.make_async_copy(ub_hbm.at[pl.ds(uvec[l], 1)], ub2.at[dst], bsem).wait()
            pltpu.make_async_copy(ib_hbm.at[pl.ds(ivec[l], 1)], ib2.at[dst], bsem).wait()
        return 0

    def bias_stage(g, _):
        rows = (g % (BCH // L)) * L + iota
        bsum = (plsc.load_gather(ub2, [rows, zeros])
                + plsc.load_gather(ib2, [rows, zeros]))
        bias_v[pl.ds(g * L, L)] = bsum
        return 0

    for c in range(NBCH):
        g0 = c * (BCH // L)
        g1 = g0 + BCH // L
        lax.fori_loop(g0, g1, bias_fire, 0)
        lax.fori_loop(g0, g1, bias_drain, 0)
        lax.fori_loop(g0, g1, bias_stage, 0)

    def emb_fire(g, _):
        uvec = u_v[pl.ds(g * L, L)]
        ivec = i_v[pl.ds(g * L, L)]
        for l in range(L):
            dst = pl.ds((g % (NG // 2)) * L + l, 1)
            pltpu.async_copy(ue_hbm.at[pl.ds(uvec[l], 1)], ue2.at[dst], sem)
            pltpu.async_copy(ie_hbm.at[pl.ds(ivec[l], 1)], ie2.at[dst], sem)
        return 0

    def emb_drain(g, _):
        uvec = u_v[pl.ds(g * L, L)]
        ivec = i_v[pl.ds(g * L, L)]
        for l in range(L):
            dst = pl.ds((g % (NG // 2)) * L + l, 1)
            pltpu.make_async_copy(ue_hbm.at[pl.ds(uvec[l], 1)], ue2.at[dst], sem).wait()
            pltpu.make_async_copy(ie_hbm.at[pl.ds(ivec[l], 1)], ie2.at[dst], sem).wait()
        return 0

    def group(g, _):
        rows = (g % (NG // 2)) * L + iota
        acc = bias_v[pl.ds(g * L, L)]
        for d in range(D):
            cols = jnp.full((L,), d, jnp.int32)
            a = plsc.load_gather(ue2, [rows, cols])
            b = plsc.load_gather(ie2, [rows, cols])
            acc = acc + a * b
        o_v[pl.ds(g * L, L)] = acc
        return 0

    for half in range(2):
        g0 = half * (NG // 2)
        g1 = g0 + NG // 2
        lax.fori_loop(g0, g1, emb_fire, 0)
        lax.fori_loop(g0, g1, emb_drain, 0)
        lax.fori_loop(g0, g1, group, 0)

    pltpu.sync_copy(o_v, out_hbm.at[pl.ds(base, BW)])


def kernel(u, i, user_emb, item_emb, user_bias, item_bias):
    return _mf_sc(u.astype(jnp.int32), i.astype(jnp.int32),
                  user_emb, item_emb, user_bias, item_bias)


# speedup vs baseline: 1.4332x; 1.4332x over previous
import functools

import jax
import jax.numpy as jnp
from jax import lax
from jax.experimental import pallas as pl
from jax.experimental.pallas import tpu as pltpu
from jax.experimental.pallas import tpu_sc as plsc

B = 16384
D = 64
NC = 2
NS = 16
NW = NC * NS
BW = B // NW
L = 16
NG = BW // L
BCH = 128
NBCH = BW // BCH
HALF = BW // 2

_mesh = plsc.VectorSubcoreMesh(core_axis_name="c", subcore_axis_name="s")


@functools.partial(
    pl.kernel,
    mesh=_mesh,
    compiler_params=pltpu.CompilerParams(needs_layout_passes=False),
    out_type=jax.ShapeDtypeStruct((B,), jnp.float32),
    scratch_types=[
        pltpu.VMEM((BW,), jnp.int32),
        pltpu.VMEM((BW,), jnp.int32),
        pltpu.VMEM((HALF, D), jnp.float32),
        pltpu.VMEM((HALF, D), jnp.float32),
        pltpu.VMEM((BW,), jnp.float32),
        pltpu.VMEM((BW,), jnp.float32),
        pltpu.VMEM((BW,), jnp.float32),
        pltpu.SemaphoreType.DMA,
        pltpu.SemaphoreType.DMA,
    ],
)
def _mf_sc(u_hbm, i_hbm, ue_hbm, ie_hbm, ub_hbm, ib_hbm, out_hbm,
           u_v, i_v, ue2, ie2, ub_v, ib_v, o_v, sem, bsem):
    wid = lax.axis_index("s") * NC + lax.axis_index("c")
    base = wid * BW

    pltpu.sync_copy(u_hbm.at[pl.ds(base, BW)], u_v)
    pltpu.sync_copy(i_hbm.at[pl.ds(base, BW)], i_v)

    bias_copies = []
    for c in range(NBCH):
        sl = pl.ds(c * BCH, BCH)
        bias_copies.append(pltpu.async_copy(ub_hbm.at[u_v.at[sl]], ub_v.at[sl], bsem))
        bias_copies.append(pltpu.async_copy(ib_hbm.at[i_v.at[sl]], ib_v.at[sl], bsem))

    iota = lax.iota(jnp.int32, L)

    def emb_fire(g, _):
        uvec = u_v[pl.ds(g * L, L)]
        ivec = i_v[pl.ds(g * L, L)]
        for l in range(L):
            dst = pl.ds((g % (NG // 2)) * L + l, 1)
            pltpu.async_copy(ue_hbm.at[pl.ds(uvec[l], 1)], ue2.at[dst], sem)
            pltpu.async_copy(ie_hbm.at[pl.ds(ivec[l], 1)], ie2.at[dst], sem)
        return 0

    def emb_drain(g, _):
        uvec = u_v[pl.ds(g * L, L)]
        ivec = i_v[pl.ds(g * L, L)]
        for l in range(L):
            dst = pl.ds((g % (NG // 2)) * L + l, 1)
            pltpu.make_async_copy(ue_hbm.at[pl.ds(uvec[l], 1)], ue2.at[dst], sem).wait()
            pltpu.make_async_copy(ie_hbm.at[pl.ds(ivec[l], 1)], ie2.at[dst], sem).wait()
        return 0

    def group(g, _):
        rows = (g % (NG // 2)) * L + iota
        acc = ub_v[pl.ds(g * L, L)] + ib_v[pl.ds(g * L, L)]
        for d in range(D):
            cols = jnp.full((L,), d, jnp.int32)
            a = plsc.load_gather(ue2, [rows, cols])
            b = plsc.load_gather(ie2, [rows, cols])
            acc = acc + a * b
        o_v[pl.ds(g * L, L)] = acc
        return 0

    for cp in bias_copies:
        cp.wait()

    for half in range(2):
        g0 = half * (NG // 2)
        g1 = g0 + NG // 2
        lax.fori_loop(g0, g1, emb_fire, 0)
        lax.fori_loop(g0, g1, emb_drain, 0)
        lax.fori_loop(g0, g1, group, 0)

    pltpu.sync_copy(o_v, out_hbm.at[pl.ds(base, BW)])


def kernel(u, i, user_emb, item_emb, user_bias, item_bias):
    n_users = user_emb.shape[0]
    n_items = item_emb.shape[0]
    return _mf_sc(u.astype(jnp.int32), i.astype(jnp.int32),
                  user_emb, item_emb,
                  user_bias.T.reshape(n_users),
                  item_bias.T.reshape(n_items))
